# Initial kernel scaffold; baseline (speedup 1.0000x reference)
import jax
import jax.numpy as jnp
from jax.experimental import pallas as pl


def kernel(x, gcn, sct, Wh, bh, Wr, br):
    raise NotImplementedError("write your pallas kernel here")


# speedup vs baseline: 1.9105x; 1.9105x over previous
import jax
import jax.numpy as jnp
from jax.experimental import pallas as pl


def _proj_kernel(x_ref, w_ref, b_ref, o_ref):
    o_ref[...] = (
        jnp.dot(x_ref[...], w_ref[...], preferred_element_type=jnp.float32)
        + b_ref[...]
    )


def _dual_mm_kernel(g_ref, s_ref, xg_ref, xs_ref, og_ref, os_ref):
    og_ref[...] = jnp.dot(g_ref[...], xg_ref[...], preferred_element_type=jnp.float32)
    os_ref[...] = jnp.dot(s_ref[...], xs_ref[...], preferred_element_type=jnp.float32)


def _level3_kernel(g_ref, s_ref, xg_ref, xs_ref, g1_ref, g2_ref, s1_ref,
                   s2_ref, wr_ref, br_ref, o_ref):
    g3 = jnp.dot(g_ref[...], xg_ref[...], preferred_element_type=jnp.float32)
    s3 = jnp.abs(jnp.dot(s_ref[...], xs_ref[...], preferred_element_type=jnp.float32))
    h = jnp.concatenate(
        [g1_ref[...], g2_ref[...], g3,
         jnp.abs(s1_ref[...]), jnp.abs(s2_ref[...]), s3], axis=1)
    h = jnp.maximum(h, 0.0)
    o_ref[...] = (
        jnp.dot(h, wr_ref[...], preferred_element_type=jnp.float32) + br_ref[...]
    )


def _single_mm_kernel(m_ref, x_ref, o_ref):
    o_ref[...] = jnp.dot(m_ref[...], x_ref[...], preferred_element_type=jnp.float32)


def _full(shape):
    return pl.BlockSpec(shape, lambda i: (0, 0))


def _rows(bm, w):
    return pl.BlockSpec((bm, w), lambda i: (i, 0))


def kernel(x, gcn, sct, Wh, bh, Wr, br):
    n, d = x.shape
    nc, _, h = Wh.shape
    out_dim = Wr.shape[1]
    wh_flat = jnp.transpose(Wh, (1, 0, 2)).reshape(d, nc * h)
    bh_flat = bh.reshape(1, nc * h)
    br2 = br.reshape(1, out_dim)

    bm_proj = 2000
    a = pl.pallas_call(
        _proj_kernel,
        grid=(n // bm_proj,),
        in_specs=[_rows(bm_proj, d), _full((d, nc * h)), _full((1, nc * h))],
        out_specs=_rows(bm_proj, nc * h),
        out_shape=jax.ShapeDtypeStruct((n, nc * h), jnp.float32),
    )(x, wh_flat, bh_flat)

    bm = 200

    def dual(xg, xs):
        wg, ws = xg.shape[1], xs.shape[1]
        return pl.pallas_call(
            _dual_mm_kernel,
            grid=(n // bm,),
            in_specs=[_rows(bm, n), _rows(bm, n), _full((n, wg)), _full((n, ws))],
            out_specs=[_rows(bm, wg), _rows(bm, ws)],
            out_shape=[jax.ShapeDtypeStruct((n, wg), jnp.float32),
                       jax.ShapeDtypeStruct((n, ws), jnp.float32)],
        )(gcn, sct, xg, xs)

    g1, s1 = dual(a[:, : 3 * h], a[:, 3 * h :])
    g2, s2 = dual(g1[:, h:], s1[:, h:])

    z = pl.pallas_call(
        _level3_kernel,
        grid=(n // bm,),
        in_specs=[_rows(bm, n), _rows(bm, n), _full((n, h)), _full((n, h)),
                  _rows(bm, h), _rows(bm, h), _rows(bm, h), _rows(bm, h),
                  _full((nc * h, out_dim)), _full((1, out_dim))],
        out_specs=_rows(bm, out_dim),
        out_shape=jax.ShapeDtypeStruct((n, out_dim), jnp.float32),
    )(gcn, sct, g2[:, h:], s2[:, h:], g1[:, :h], g2[:, :h], s1[:, :h],
      s2[:, :h], Wr, br2)

    out = pl.pallas_call(
        _single_mm_kernel,
        grid=(n // bm,),
        in_specs=[_rows(bm, n), _full((n, out_dim))],
        out_specs=_rows(bm, out_dim),
        out_shape=jax.ShapeDtypeStruct((n, out_dim), jnp.float32),
    )(gcn, z)
    return out


# speedup vs baseline: 2.2740x; 1.1903x over previous
import jax
import jax.numpy as jnp
from jax.experimental import pallas as pl


def _proj_kernel(x_ref, w_ref, b_ref, o_ref):
    o_ref[...] = (
        jnp.dot(x_ref[...], w_ref[...], preferred_element_type=jnp.float32)
        + b_ref[...]
    )


def _dual_mm_cast_kernel(g_ref, s_ref, xg_ref, xs_ref, og_ref, os_ref,
                         gb_ref, sb_ref):
    g, s = g_ref[...], s_ref[...]
    og_ref[...] = jnp.dot(g, xg_ref[...], preferred_element_type=jnp.float32)
    os_ref[...] = jnp.dot(s, xs_ref[...], preferred_element_type=jnp.float32)
    gb_ref[...] = g.astype(jnp.bfloat16)
    sb_ref[...] = s.astype(jnp.bfloat16)


def _dual_mm_kernel(g_ref, s_ref, xg_ref, xs_ref, og_ref, os_ref):
    og_ref[...] = jnp.dot(g_ref[...], xg_ref[...], preferred_element_type=jnp.float32)
    os_ref[...] = jnp.dot(s_ref[...], xs_ref[...], preferred_element_type=jnp.float32)


def _level3_kernel(g_ref, s_ref, xg_ref, xs_ref, g1_ref, g2_ref, s1_ref,
                   s2_ref, wr_ref, br_ref, o_ref):
    g3 = jnp.dot(g_ref[...], xg_ref[...], preferred_element_type=jnp.float32)
    s3 = jnp.abs(jnp.dot(s_ref[...], xs_ref[...], preferred_element_type=jnp.float32))
    h = jnp.concatenate(
        [g1_ref[...], g2_ref[...], g3,
         jnp.abs(s1_ref[...]), jnp.abs(s2_ref[...]), s3], axis=1)
    h = jnp.maximum(h, 0.0)
    o_ref[...] = (
        jnp.dot(h, wr_ref[...], preferred_element_type=jnp.float32) + br_ref[...]
    )


def _single_mm_kernel(m_ref, x_ref, o_ref):
    o_ref[...] = jnp.dot(m_ref[...], x_ref[...], preferred_element_type=jnp.float32)


def _full(shape):
    return pl.BlockSpec(shape, lambda i: (0, 0))


def _rows(bm, w):
    return pl.BlockSpec((bm, w), lambda i: (i, 0))


def kernel(x, gcn, sct, Wh, bh, Wr, br):
    n, d = x.shape
    nc, _, h = Wh.shape
    out_dim = Wr.shape[1]
    wh_flat = jnp.transpose(Wh, (1, 0, 2)).reshape(d, nc * h)
    bh_flat = bh.reshape(1, nc * h)
    br2 = br.reshape(1, out_dim)

    bm_proj = 2000
    a = pl.pallas_call(
        _proj_kernel,
        grid=(n // bm_proj,),
        in_specs=[_rows(bm_proj, d), _full((d, nc * h)), _full((1, nc * h))],
        out_specs=_rows(bm_proj, nc * h),
        out_shape=jax.ShapeDtypeStruct((n, nc * h), jnp.float32),
    )(x, wh_flat, bh_flat)

    bm1 = 200
    xg1, xs1 = a[:, : 3 * h], a[:, 3 * h :]
    w1 = 3 * h
    g1, s1, gcn_b, sct_b = pl.pallas_call(
        _dual_mm_cast_kernel,
        grid=(n // bm1,),
        in_specs=[_rows(bm1, n), _rows(bm1, n), _full((n, w1)), _full((n, w1))],
        out_specs=[_rows(bm1, w1), _rows(bm1, w1), _rows(bm1, n), _rows(bm1, n)],
        out_shape=[jax.ShapeDtypeStruct((n, w1), jnp.float32),
                   jax.ShapeDtypeStruct((n, w1), jnp.float32),
                   jax.ShapeDtypeStruct((n, n), jnp.bfloat16),
                   jax.ShapeDtypeStruct((n, n), jnp.bfloat16)],
    )(gcn, sct, xg1, xs1)

    bm = 400

    def dual(xg, xs):
        wg, ws = xg.shape[1], xs.shape[1]
        return pl.pallas_call(
            _dual_mm_kernel,
            grid=(n // bm,),
            in_specs=[_rows(bm, n), _rows(bm, n), _full((n, wg)), _full((n, ws))],
            out_specs=[_rows(bm, wg), _rows(bm, ws)],
            out_shape=[jax.ShapeDtypeStruct((n, wg), jnp.float32),
                       jax.ShapeDtypeStruct((n, ws), jnp.float32)],
        )(gcn_b, sct_b, xg, xs)

    g2, s2 = dual(g1[:, h:].astype(jnp.bfloat16), s1[:, h:].astype(jnp.bfloat16))

    z = pl.pallas_call(
        _level3_kernel,
        grid=(n // bm,),
        in_specs=[_rows(bm, n), _rows(bm, n), _full((n, h)), _full((n, h)),
                  _rows(bm, h), _rows(bm, h), _rows(bm, h), _rows(bm, h),
                  _full((nc * h, out_dim)), _full((1, out_dim))],
        out_specs=_rows(bm, out_dim),
        out_shape=jax.ShapeDtypeStruct((n, out_dim), jnp.float32),
    )(gcn_b, sct_b, g2[:, h:].astype(jnp.bfloat16), s2[:, h:].astype(jnp.bfloat16),
      g1[:, :h], g2[:, :h], s1[:, :h], s2[:, :h], Wr, br2)

    out = pl.pallas_call(
        _single_mm_kernel,
        grid=(n // bm,),
        in_specs=[_rows(bm, n), _full((n, out_dim))],
        out_specs=_rows(bm, out_dim),
        out_shape=jax.ShapeDtypeStruct((n, out_dim), jnp.float32),
    )(gcn_b, z.astype(jnp.bfloat16))
    return out
